# baseline (speedup 1.0000x reference)
import functools

import jax
import jax.numpy as jnp
from jax.experimental import pallas as pl
from jax.experimental.pallas import tpu as pltpu


def _next_pow2(x):
    p = 1
    while p < x:
        p *= 2
    return p


def _two_word_gt(a_v, b_v, a_i, b_i):
    return (a_v > b_v) | ((a_v == b_v) & (a_i < b_i))


def _cmpex_roll(vals, idxs, stride, size, half_lanes):
    S, W = vals.shape
    sub = jax.lax.broadcasted_iota(jnp.int32, (S, W), 0)
    first = (sub & stride) == 0
    p_v = jnp.where(first, pltpu.roll(vals, S - stride, 0), pltpu.roll(vals, stride, 0))
    p_i = jnp.where(first, pltpu.roll(idxs, S - stride, 0), pltpu.roll(idxs, stride, 0))
    i_win = _two_word_gt(vals, p_v, idxs, p_i)
    dir_desc = None if size is None else ((sub & size) == 0)
    if half_lanes is not None:
        lane_desc = jax.lax.broadcasted_iota(jnp.int32, (S, W), 1) < half_lanes
        dir_desc = lane_desc if dir_desc is None else (dir_desc == lane_desc)
    keep_winner = first if dir_desc is None else (first == dir_desc)
    new_v = jnp.where(i_win == keep_winner, vals, p_v)
    new_i = jnp.where(i_win == keep_winner, idxs, p_i)
    return new_v, new_i


def _cmpex(vals, idxs, stride, size, half_lanes):
    if stride < 8:
        return _cmpex_roll(vals, idxs, stride, size, half_lanes)
    S, W = vals.shape

    def partner(x):
        x4 = x.reshape(S // (2 * stride), 2, stride, W)
        p4 = jnp.concatenate([x4[:, 1:2], x4[:, 0:1]], axis=1)
        return p4.reshape(S, W)

    p_v = partner(vals)
    p_i = partner(idxs)
    sub = jax.lax.broadcasted_iota(jnp.int32, (S, W), 0)
    first = (sub & stride) == 0
    i_win = _two_word_gt(vals, p_v, idxs, p_i)
    dir_desc = None if size is None else ((sub & size) == 0)
    if half_lanes is not None:
        lane_desc = jax.lax.broadcasted_iota(jnp.int32, (S, W), 1) < half_lanes
        dir_desc = lane_desc if dir_desc is None else (dir_desc == lane_desc)
    keep_winner = first if dir_desc is None else (first == dir_desc)
    new_v = jnp.where(i_win == keep_winner, vals, p_v)
    new_i = jnp.where(i_win == keep_winner, idxs, p_i)
    return new_v, new_i


def _topk_body(K, N, C, SP, x_ref, poly_ref, scale_ref, scores_ref, labels_ref, polys_ref):
    s = jax.nn.sigmoid(x_ref[0])
    if SP > N:
        s = jnp.concatenate([s, jnp.full((SP - N, C), -1.0, jnp.float32)], axis=0)
    idx = (jax.lax.broadcasted_iota(jnp.int32, (SP, C), 0) * C
           + jax.lax.broadcasted_iota(jnp.int32, (SP, C), 1))

    size = 2
    while size <= SP:
        stride = size // 2
        while stride >= 1:
            s, idx = _cmpex(s, idx, stride, size, C // 2 if C > 1 else None)
            stride //= 2
        size *= 2

    W = C
    while W > 1:
        W //= 2
        a_v, b_v = s[:, :W], s[:, W:2 * W]
        a_i, b_i = idx[:, :W], idx[:, W:2 * W]
        win = _two_word_gt(a_v, b_v, a_i, b_i)
        s = jnp.where(win, a_v, b_v)
        idx = jnp.where(win, a_i, b_i)
        half = W // 2 if W > 1 else None
        stride = SP // 2
        while stride >= 1:
            s, idx = _cmpex(s, idx, stride, None, half)
            stride //= 2

    s_lane = s.reshape(1, SP)
    i_lane = idx.reshape(1, SP)
    scores_ref[0, 0, :] = s_lane[0, :K]
    labels_ref[0, 0, :] = i_lane[0, :K] % C

    q_col = idx // C
    oh = (q_col == jax.lax.broadcasted_iota(jnp.int32, (SP, N), 1)).astype(jnp.float32)
    polys = jax.lax.dot_general(
        oh, poly_ref[0],
        dimension_numbers=(((1,), (0,)), ((), ())),
        preferred_element_type=jnp.float32,
    )
    polys_ref[0] = (polys * scale_ref[0, 0, :])[:K]


def kernel(pred_logits, pred_polys, target_sizes):
    B, N, C = pred_logits.shape
    K = N
    D = pred_polys.shape[-1]
    SP = _next_pow2(N)
    img_h = target_sizes[:, 0].astype(jnp.float32)
    img_w = target_sizes[:, 1].astype(jnp.float32)
    scale = jnp.stack([img_w, img_h] * (D // 2), axis=1).reshape(B, 1, D)

    out = pl.pallas_call(
        functools.partial(_topk_body, K, N, C, SP),
        grid=(B,),
        in_specs=[
            pl.BlockSpec((1, N, C), lambda b: (b, 0, 0)),
            pl.BlockSpec((1, N, D), lambda b: (b, 0, 0)),
            pl.BlockSpec((1, 1, D), lambda b: (b, 0, 0)),
        ],
        out_specs=[
            pl.BlockSpec((1, 1, K), lambda b: (b, 0, 0)),
            pl.BlockSpec((1, 1, K), lambda b: (b, 0, 0)),
            pl.BlockSpec((1, K, D), lambda b: (b, 0, 0)),
        ],
        out_shape=[
            jax.ShapeDtypeStruct((B, 1, K), jnp.float32),
            jax.ShapeDtypeStruct((B, 1, K), jnp.int32),
            jax.ShapeDtypeStruct((B, K, D), jnp.float32),
        ],
    )(pred_logits, pred_polys, scale)
    scores, labels, polys = out
    return scores.reshape(B, K), labels.reshape(B, K), polys


# speedup vs baseline: 1.4737x; 1.4737x over previous
import functools

import jax
import jax.numpy as jnp
from jax.experimental import pallas as pl
from jax.experimental.pallas import tpu as pltpu


def _next_pow2(x):
    p = 1
    while p < x:
        p *= 2
    return p


def _two_word_gt(a_v, b_v, a_i, b_i):
    return (a_v > b_v) | ((a_v == b_v) & (a_i < b_i))


def _stage(vals, idxs, stride, fms, dir1=None, dirL=None, flip=False):
    S, W = vals.shape
    if stride < 8:
        fm = fms[stride]
        p_v = jnp.where(fm, pltpu.roll(vals, S - stride, 0), pltpu.roll(vals, stride, 0))
        p_i = jnp.where(fm, pltpu.roll(idxs, S - stride, 0), pltpu.roll(idxs, stride, 0))
        i_win = _two_word_gt(vals, p_v, idxs, p_i)
        if dir1 is not None:
            kw = fm == dir1
        elif dirL is not None:
            kw = fm == dirL
        else:
            kw = fm if not flip else ~fm
        keep = i_win == kw
        return jnp.where(keep, vals, p_v), jnp.where(keep, idxs, p_i)

    G = S // (2 * stride)
    v4 = vals.reshape(G, 2, stride, W)
    i4 = idxs.reshape(G, 2, stride, W)
    a_v, b_v = v4[:, 0], v4[:, 1]
    a_i, b_i = i4[:, 0], i4[:, 1]
    a_win = _two_word_gt(a_v, b_v, a_i, b_i)
    swap = False
    if dir1 is not None:
        keep = a_win == dir1.reshape(G, 2, stride, 1)[:, 0]
    elif dirL is not None:
        keep = a_win == dirL.reshape(1, 1, W)
    else:
        keep = a_win
        swap = flip
    if swap:
        new_a_v = jnp.where(keep, b_v, a_v)
        new_b_v = jnp.where(keep, a_v, b_v)
        new_a_i = jnp.where(keep, b_i, a_i)
        new_b_i = jnp.where(keep, a_i, b_i)
    else:
        new_a_v = jnp.where(keep, a_v, b_v)
        new_b_v = jnp.where(keep, b_v, a_v)
        new_a_i = jnp.where(keep, a_i, b_i)
        new_b_i = jnp.where(keep, b_i, a_i)
    vals = jnp.concatenate([new_a_v[:, None], new_b_v[:, None]], axis=1).reshape(S, W)
    idxs = jnp.concatenate([new_a_i[:, None], new_b_i[:, None]], axis=1).reshape(S, W)
    return vals, idxs


def _leaf_sort(vals, idxs, SP, fms, sub1, asc):
    size = 2
    while size <= SP:
        if size >= SP:
            dir1 = None
        else:
            dir1 = ((sub1 & size) == 0) != asc
        stride = size // 2
        while stride >= 1:
            vals, idxs = _stage(vals, idxs, stride, fms, dir1=dir1, flip=asc)
            stride //= 2
        size *= 2
    return vals, idxs


def _topk_body(K, N, C, SP, x_ref, poly_ref, scale_ref, scores_ref, labels_ref, polys_ref):
    s = jax.nn.sigmoid(x_ref[0])
    if SP > N:
        s = jnp.concatenate([s, jnp.full((SP - N, C), -1.0, jnp.float32)], axis=0)
    idx = (jax.lax.broadcasted_iota(jnp.int32, (SP, C), 0) * C
           + jax.lax.broadcasted_iota(jnp.int32, (SP, C), 1))

    sub1 = jax.lax.broadcasted_iota(jnp.int32, (SP, 1), 0)
    fms = {st: (sub1 & st) == 0 for st in (1, 2, 4) if st < 8 <= SP}

    H = C // 2
    v_l, i_l = _leaf_sort(s[:, :H], idx[:, :H], SP, fms, sub1, asc=False)
    v_r, i_r = _leaf_sort(s[:, H:], idx[:, H:], SP, fms, sub1, asc=True)
    s = jnp.concatenate([v_l, v_r], axis=1)
    idx = jnp.concatenate([i_l, i_r], axis=1)

    W = C
    while W > 1:
        W //= 2
        a_v, b_v = s[:, :W], s[:, W:2 * W]
        a_i, b_i = idx[:, :W], idx[:, W:2 * W]
        win = _two_word_gt(a_v, b_v, a_i, b_i)
        s = jnp.where(win, a_v, b_v)
        idx = jnp.where(win, a_i, b_i)
        if W > 1:
            dirL = jax.lax.broadcasted_iota(jnp.int32, (1, W), 1) < (W // 2)
        else:
            dirL = None
        stride = SP // 2
        while stride >= 1:
            s, idx = _stage(s, idx, stride, fms, dirL=dirL)
            stride //= 2

    s_lane = s.reshape(1, SP)
    i_lane = idx.reshape(1, SP)
    scores_ref[0, 0, :] = s_lane[0, :K]
    labels_ref[0, 0, :] = i_lane[0, :K] % C

    q_col = idx // C
    oh = (q_col == jax.lax.broadcasted_iota(jnp.int32, (SP, N), 1)).astype(jnp.float32)
    polys = jax.lax.dot_general(
        oh, poly_ref[0],
        dimension_numbers=(((1,), (0,)), ((), ())),
        preferred_element_type=jnp.float32,
    )
    polys_ref[0] = (polys * scale_ref[0, 0, :])[:K]


def kernel(pred_logits, pred_polys, target_sizes):
    B, N, C = pred_logits.shape
    K = N
    D = pred_polys.shape[-1]
    SP = _next_pow2(N)
    img_h = target_sizes[:, 0].astype(jnp.float32)
    img_w = target_sizes[:, 1].astype(jnp.float32)
    scale = jnp.stack([img_w, img_h] * (D // 2), axis=1).reshape(B, 1, D)

    out = pl.pallas_call(
        functools.partial(_topk_body, K, N, C, SP),
        grid=(B,),
        in_specs=[
            pl.BlockSpec((1, N, C), lambda b: (b, 0, 0)),
            pl.BlockSpec((1, N, D), lambda b: (b, 0, 0)),
            pl.BlockSpec((1, 1, D), lambda b: (b, 0, 0)),
        ],
        out_specs=[
            pl.BlockSpec((1, 1, K), lambda b: (b, 0, 0)),
            pl.BlockSpec((1, 1, K), lambda b: (b, 0, 0)),
            pl.BlockSpec((1, K, D), lambda b: (b, 0, 0)),
        ],
        out_shape=[
            jax.ShapeDtypeStruct((B, 1, K), jnp.float32),
            jax.ShapeDtypeStruct((B, 1, K), jnp.int32),
            jax.ShapeDtypeStruct((B, K, D), jnp.float32),
        ],
    )(pred_logits, pred_polys, scale)
    scores, labels, polys = out
    return scores.reshape(B, K), labels.reshape(B, K), polys


# speedup vs baseline: 2.0125x; 1.3657x over previous
import functools

import jax
import jax.numpy as jnp
from jax.experimental import pallas as pl
from jax.experimental.pallas import tpu as pltpu


def _next_pow2(x):
    p = 1
    while p < x:
        p *= 2
    return p


def _two_word_gt(a_v, b_v, a_i, b_i):
    return (a_v > b_v) | ((a_v == b_v) & (a_i < b_i))


def _stage(vals, idxs, stride, fms, dir1=None, dirL=None, flip=False):
    S, W = vals.shape
    if stride < 8:
        fm = fms[stride]
        p_v = jnp.where(fm, pltpu.roll(vals, S - stride, 0), pltpu.roll(vals, stride, 0))
        p_i = jnp.where(fm, pltpu.roll(idxs, S - stride, 0), pltpu.roll(idxs, stride, 0))
        i_win = _two_word_gt(vals, p_v, idxs, p_i)
        if dir1 is not None:
            kw = fm == dir1
        elif dirL is not None:
            kw = fm == dirL
        else:
            kw = fm if not flip else ~fm
        keep = i_win == kw
        return jnp.where(keep, vals, p_v), jnp.where(keep, idxs, p_i)

    G = S // (2 * stride)
    v4 = vals.reshape(G, 2, stride, W)
    i4 = idxs.reshape(G, 2, stride, W)
    a_v, b_v = v4[:, 0], v4[:, 1]
    a_i, b_i = i4[:, 0], i4[:, 1]
    a_win = _two_word_gt(a_v, b_v, a_i, b_i)
    swap = False
    if dir1 is not None:
        keep = a_win == dir1.reshape(G, 2, stride, 1)[:, 0]
    elif dirL is not None:
        keep = a_win == dirL.reshape(1, 1, W)
    else:
        keep = a_win
        swap = flip
    if swap:
        new_a_v = jnp.where(keep, b_v, a_v)
        new_b_v = jnp.where(keep, a_v, b_v)
        new_a_i = jnp.where(keep, b_i, a_i)
        new_b_i = jnp.where(keep, a_i, b_i)
    else:
        new_a_v = jnp.where(keep, a_v, b_v)
        new_b_v = jnp.where(keep, b_v, a_v)
        new_a_i = jnp.where(keep, a_i, b_i)
        new_b_i = jnp.where(keep, b_i, a_i)
    vals = jnp.concatenate([new_a_v[:, None], new_b_v[:, None]], axis=1).reshape(S, W)
    idxs = jnp.concatenate([new_a_i[:, None], new_b_i[:, None]], axis=1).reshape(S, W)
    return vals, idxs


def _leaf_sort(vals, idxs, SP, fms, sub1, asc):
    size = 2
    while size <= SP:
        if size >= SP:
            dir1 = None
        else:
            dir1 = ((sub1 & size) == 0) != asc
        stride = size // 2
        while stride >= 1:
            vals, idxs = _stage(vals, idxs, stride, fms, dir1=dir1, flip=asc)
            stride //= 2
        size *= 2
    return vals, idxs


def _merge_level(s, idx, SP, fms, dirL):
    W = s.shape[1] // 2
    a_v, b_v = s[:, :W], s[:, W:]
    a_i, b_i = idx[:, :W], idx[:, W:]
    win = _two_word_gt(a_v, b_v, a_i, b_i)
    s = jnp.where(win, a_v, b_v)
    idx = jnp.where(win, a_i, b_i)
    stride = SP // 2
    while stride >= 1:
        s, idx = _stage(s, idx, stride, fms, dirL=dirL)
        stride //= 2
    return s, idx


def _topk_body(K, N, C, SP, R, x_ref, poly_ref, scale_ref, perm_ref,
               scores_ref, labels_ref, polys_ref):
    sub1 = jax.lax.broadcasted_iota(jnp.int32, (SP, 1), 0)
    fms = {st: (sub1 & st) == 0 for st in (1, 2, 4) if st < SP}
    H = C // 2

    cols_v, cols_i = [], []
    for r in range(R):
        s = jax.nn.sigmoid(x_ref[r])
        if SP > N:
            s = jnp.concatenate([s, jnp.full((SP - N, C), -1.0, jnp.float32)], axis=0)
        idx = (jax.lax.broadcasted_iota(jnp.int32, (SP, C), 0) * C
               + jax.lax.broadcasted_iota(jnp.int32, (SP, C), 1)).astype(jnp.float32)
        v_l, i_l = _leaf_sort(s[:, :H], idx[:, :H], SP, fms, sub1, asc=False)
        v_r, i_r = _leaf_sort(s[:, H:], idx[:, H:], SP, fms, sub1, asc=True)
        s = jnp.concatenate([v_l, v_r], axis=1)
        idx = jnp.concatenate([i_l, i_r], axis=1)
        dirL = jax.lax.broadcasted_iota(jnp.int32, (1, H), 1) < (H // 2)
        s, idx = _merge_level(s, idx, SP, fms, dirL if H > 1 else None)
        cols_v.append(s)
        cols_i.append(idx)

    if R > 1:
        Xv = jnp.concatenate(cols_v, axis=1)
        Xi = jnp.concatenate(cols_i, axis=1)
        P = perm_ref[...]
        Xv = jax.lax.dot_general(Xv, P, dimension_numbers=(((1,), (0,)), ((), ())),
                                 preferred_element_type=jnp.float32)
        Xi = jax.lax.dot_general(Xi, P, dimension_numbers=(((1,), (0,)), ((), ())),
                                 preferred_element_type=jnp.float32)
    else:
        Xv, Xi = cols_v[0], cols_i[0]

    cw = H
    while cw > 1:
        cw //= 2
        Wb = cw * R
        dirL = (jax.lax.broadcasted_iota(jnp.int32, (1, Wb), 1) < (Wb // 2)
                if cw > 1 else None)
        Xv, Xi = _merge_level(Xv, Xi, SP, fms, dirL)

    for r in range(R):
        v_lane = Xv[:, r:r + 1].reshape(1, SP)
        i_lane = Xi[:, r:r + 1].reshape(1, SP).astype(jnp.int32)
        scores_ref[r, 0, :] = v_lane[0, :K]
        labels_ref[r, 0, :] = i_lane[0, :K] % C
        q_col = Xi[:, r:r + 1].astype(jnp.int32) // C
        oh = (q_col == jax.lax.broadcasted_iota(jnp.int32, (SP, N), 1)).astype(jnp.float32)
        polys = jax.lax.dot_general(
            oh, poly_ref[r],
            dimension_numbers=(((1,), (0,)), ((), ())),
            preferred_element_type=jnp.float32,
        )
        polys_ref[r] = (polys * scale_ref[r, 0, :])[:K]


def kernel(pred_logits, pred_polys, target_sizes):
    B, N, C = pred_logits.shape
    K = N
    D = pred_polys.shape[-1]
    SP = _next_pow2(N)
    R = 8 if B % 8 == 0 else 1
    H = C // 2
    PW = H * R
    img_h = target_sizes[:, 0].astype(jnp.float32)
    img_w = target_sizes[:, 1].astype(jnp.float32)
    scale = jnp.stack([img_w, img_h] * (D // 2), axis=1).reshape(B, 1, D)

    j = jnp.arange(PW)
    k_of_j = (j % H) * R + j // H
    perm = (k_of_j[:, None] == jnp.arange(PW)[None, :]).astype(jnp.float32)

    out = pl.pallas_call(
        functools.partial(_topk_body, K, N, C, SP, R),
        grid=(B // R,),
        in_specs=[
            pl.BlockSpec((R, N, C), lambda b: (b, 0, 0)),
            pl.BlockSpec((R, N, D), lambda b: (b, 0, 0)),
            pl.BlockSpec((R, 1, D), lambda b: (b, 0, 0)),
            pl.BlockSpec((PW, PW), lambda b: (0, 0)),
        ],
        out_specs=[
            pl.BlockSpec((R, 1, K), lambda b: (b, 0, 0)),
            pl.BlockSpec((R, 1, K), lambda b: (b, 0, 0)),
            pl.BlockSpec((R, K, D), lambda b: (b, 0, 0)),
        ],
        out_shape=[
            jax.ShapeDtypeStruct((B, 1, K), jnp.float32),
            jax.ShapeDtypeStruct((B, 1, K), jnp.int32),
            jax.ShapeDtypeStruct((B, K, D), jnp.float32),
        ],
    )(pred_logits, pred_polys, scale, perm)
    scores, labels, polys = out
    return scores.reshape(B, K), labels.reshape(B, K), polys
